# baseline (speedup 1.0000x reference)
import functools

import jax
import jax.numpy as jnp
from jax import lax
from jax.experimental import pallas as pl
from jax.experimental.pallas import tpu as pltpu
from jax.experimental.pallas import tpu_sc as plsc

N_POSITIONS = 512
N_EMBD = 128

_B = 4096 * 200
_NC = 2
_NS = 16
_NW = _NC * _NS
_BPW = _B // _NW
_IDXC = 128
_SUB = 2
_CHUNK = _IDXC * _SUB
_NCHUNK = _BPW // _CHUNK
_NBUF = 3
_NGROUP = _NCHUNK // _NBUF
_NTAIL = _NCHUNK - _NGROUP * _NBUF

_mesh = plsc.VectorSubcoreMesh(core_axis_name="c", subcore_axis_name="s")


@functools.partial(
    pl.kernel,
    mesh=_mesh,
    out_type=jax.ShapeDtypeStruct((_B, N_EMBD), jnp.float32),
    scratch_types=[
        pltpu.VMEM_SHARED((N_POSITIONS, N_EMBD), jnp.float32),
        pltpu.VMEM((_BPW,), jnp.int32),
        pltpu.VMEM((_NBUF, _CHUNK, N_EMBD), jnp.float32),
        pltpu.SemaphoreType.DMA,
        pltpu.SemaphoreType.DMA((_NBUF,)),
        pltpu.SemaphoreType.DMA((_NBUF,)),
    ],
)
def _gather_kernel(ids_hbm, table_hbm, out_hbm, tab_s, idx_v, rows_v,
                   isem, gsem, wsem):
    wid = lax.axis_index("s") * _NC + lax.axis_index("c")
    base = wid * _BPW

    def fire_gathers(c, b):
        for r in range(_SUB):
            pltpu.async_copy(
                tab_s.at[idx_v.at[pl.ds(c * _CHUNK + r * _IDXC, _IDXC)]],
                rows_v.at[b, pl.ds(r * _IDXC, _IDXC)],
                gsem.at[b])

    def wait_gathers(c, b):
        for r in range(_SUB):
            pltpu.make_async_copy(
                tab_s.at[idx_v.at[pl.ds(c * _CHUNK + r * _IDXC, _IDXC)]],
                rows_v.at[b, pl.ds(r * _IDXC, _IDXC)],
                gsem.at[b]).wait()

    def fire_write(c, b):
        pltpu.async_copy(
            rows_v.at[b], out_hbm.at[pl.ds(base + c * _CHUNK, _CHUNK)],
            wsem.at[b])

    def wait_write(c, b):
        pltpu.make_async_copy(
            rows_v.at[b], out_hbm.at[pl.ds(base + c * _CHUNK, _CHUNK)],
            wsem.at[b]).wait()

    pltpu.async_copy(ids_hbm.at[pl.ds(base, _BPW)], idx_v, isem)

    @pl.when(lax.axis_index("s") == 0)
    def _():
        pltpu.sync_copy(table_hbm, tab_s)

    plsc.subcore_barrier()
    pltpu.make_async_copy(ids_hbm.at[pl.ds(base, _BPW)], idx_v, isem).wait()

    for b in range(_NBUF):
        fire_gathers(b, b)

    def body(g, carry):
        c0 = g * _NBUF
        for b in range(_NBUF):
            wait_gathers(c0 + b, b)
            fire_write(c0 + b, b)

        @pl.when(g + 1 < _NGROUP)
        def _():
            for b in range(_NBUF):
                wait_write(c0 + b, b)
                fire_gathers(c0 + _NBUF + b, b)

        return carry

    lax.fori_loop(0, _NGROUP, body, 0)

    last0 = (_NGROUP - 1) * _NBUF
    for t in range(_NTAIL):
        wait_write(last0 + t, t)
        fire_gathers(_NGROUP * _NBUF + t, t)
    for t in range(_NTAIL):
        wait_gathers(_NGROUP * _NBUF + t, t)
        fire_write(_NGROUP * _NBUF + t, t)
    for b in range(_NTAIL, _NBUF):
        wait_write(last0 + b, b)
    for t in range(_NTAIL):
        wait_write(_NGROUP * _NBUF + t, t)


def kernel(position_ids, pe):
    ids_flat = jnp.reshape(position_ids, (_B,))
    out = _gather_kernel(ids_flat, pe)
    return jnp.reshape(out, (*position_ids.shape, N_EMBD))


# speedup vs baseline: 1.0471x; 1.0471x over previous
import functools

import jax
import jax.numpy as jnp
from jax import lax
from jax.experimental import pallas as pl
from jax.experimental.pallas import tpu as pltpu
from jax.experimental.pallas import tpu_sc as plsc

N_POSITIONS = 512
N_EMBD = 128

_B = 4096 * 200
_NC = 2
_NS = 16
_NW = _NC * _NS
_BPW = _B // _NW
_IDXC = 64
_SUB = 1
_CHUNK = _IDXC * _SUB
_NCHUNK = _BPW // _CHUNK
_NBUF = 12
_NGROUP = _NCHUNK // _NBUF
_NTAIL = _NCHUNK - _NGROUP * _NBUF

_mesh = plsc.VectorSubcoreMesh(core_axis_name="c", subcore_axis_name="s")


@functools.partial(
    pl.kernel,
    mesh=_mesh,
    out_type=jax.ShapeDtypeStruct((_B, N_EMBD), jnp.float32),
    scratch_types=[
        pltpu.VMEM_SHARED((N_POSITIONS, N_EMBD), jnp.float32),
        pltpu.VMEM((_BPW,), jnp.int32),
        pltpu.VMEM((_NBUF, _CHUNK, N_EMBD), jnp.float32),
        pltpu.SemaphoreType.DMA,
        pltpu.SemaphoreType.DMA((_NBUF,)),
        pltpu.SemaphoreType.DMA((_NBUF,)),
    ],
)
def _gather_kernel(ids_hbm, table_hbm, out_hbm, tab_s, idx_v, rows_v,
                   isem, gsem, wsem):
    wid = lax.axis_index("s") * _NC + lax.axis_index("c")
    base = wid * _BPW

    def fire_gathers(c, b):
        for r in range(_SUB):
            pltpu.async_copy(
                tab_s.at[idx_v.at[pl.ds(c * _CHUNK + r * _IDXC, _IDXC)]],
                rows_v.at[b, pl.ds(r * _IDXC, _IDXC)],
                gsem.at[b])

    def wait_gathers(c, b):
        for r in range(_SUB):
            pltpu.make_async_copy(
                tab_s.at[idx_v.at[pl.ds(c * _CHUNK + r * _IDXC, _IDXC)]],
                rows_v.at[b, pl.ds(r * _IDXC, _IDXC)],
                gsem.at[b]).wait()

    def fire_write(c, b):
        pltpu.async_copy(
            rows_v.at[b], out_hbm.at[pl.ds(base + c * _CHUNK, _CHUNK)],
            wsem.at[b])

    def wait_write(c, b):
        pltpu.make_async_copy(
            rows_v.at[b], out_hbm.at[pl.ds(base + c * _CHUNK, _CHUNK)],
            wsem.at[b]).wait()

    pltpu.async_copy(ids_hbm.at[pl.ds(base, _BPW)], idx_v, isem)

    @pl.when(lax.axis_index("s") == 0)
    def _():
        pltpu.sync_copy(table_hbm, tab_s)

    plsc.subcore_barrier()
    pltpu.make_async_copy(ids_hbm.at[pl.ds(base, _BPW)], idx_v, isem).wait()

    for b in range(_NBUF):
        fire_gathers(b, b)

    def body(g, carry):
        c0 = g * _NBUF
        for b in range(_NBUF):
            wait_gathers(c0 + b, b)
            fire_write(c0 + b, b)

        @pl.when(g + 1 < _NGROUP)
        def _():
            for b in range(_NBUF):
                wait_write(c0 + b, b)
                fire_gathers(c0 + _NBUF + b, b)

        return carry

    lax.fori_loop(0, _NGROUP, body, 0)

    last0 = (_NGROUP - 1) * _NBUF
    for t in range(_NTAIL):
        wait_write(last0 + t, t)
        fire_gathers(_NGROUP * _NBUF + t, t)
    for t in range(_NTAIL):
        wait_gathers(_NGROUP * _NBUF + t, t)
        fire_write(_NGROUP * _NBUF + t, t)
    for b in range(_NTAIL, _NBUF):
        wait_write(last0 + b, b)
    for t in range(_NTAIL):
        wait_write(_NGROUP * _NBUF + t, t)


def kernel(position_ids, pe):
    ids_flat = jnp.reshape(position_ids, (_B,))
    out = _gather_kernel(ids_flat, pe)
    return jnp.reshape(out, (*position_ids.shape, N_EMBD))
